# baseline (speedup 1.0000x reference)
import functools

import numpy as np
import jax
import jax.numpy as jnp
from jax import lax
from jax.experimental import pallas as pl
from jax.experimental.pallas import tpu as pltpu
from jax.experimental.pallas import tpu_sc as plsc

F32 = jnp.float32
NBLK = 1000
EBLK = 3200
NW = 32
CH = 128


def _silu(x):
    return x * jax.nn.sigmoid(x)


def _dot(a, b):
    return jnp.dot(a, b, preferred_element_type=F32, precision='highest')


def _node_init(coords, atoms2, masses, emb, w1p, w1e, w1m, b1, w2, b2):
    n = coords.shape[0]
    grid = (n // NBLK,)

    def body(c_ref, a_ref, ms_ref, emb_ref, w1p_ref, w1e_ref, w1m_ref,
             b1_ref, w2_ref, b2_ref, o_ref):
        pos = jnp.abs(c_ref[...])
        at = a_ref[...]
        oh = (at == lax.broadcasted_iota(jnp.int32, (NBLK, 90), 1)).astype(F32)
        aemb = _dot(oh, emb_ref[...])
        x = (_dot(pos, w1p_ref[...]) + _dot(aemb, w1e_ref[...])
             + _dot(ms_ref[...], w1m_ref[...]) + b1_ref[...])
        o_ref[...] = _dot(_silu(x), w2_ref[...]) + b2_ref[...]

    full = lambda s: pl.BlockSpec(s, lambda i: tuple(0 for _ in s))
    return pl.pallas_call(
        body,
        grid=grid,
        in_specs=[
            pl.BlockSpec((NBLK, 3), lambda i: (i, 0)),
            pl.BlockSpec((NBLK, 1), lambda i: (i, 0)),
            pl.BlockSpec((NBLK, 1), lambda i: (i, 0)),
            full(emb.shape), full(w1p.shape), full(w1e.shape),
            full(w1m.shape), full(b1.shape), full(w2.shape), full(b2.shape),
        ],
        out_specs=pl.BlockSpec((NBLK, 64), lambda i: (i, 0)),
        out_shape=jax.ShapeDtypeStruct((n, 64), F32),
    )(coords, atoms2, masses, emb, w1p, w1e, w1m, b1, w2, b2)


def _cond(x256, moments, w1a, w1m, b1, w2, b2, gws, bws, bgs, bbs):
    bsz = x256.shape[0]
    nl4 = gws.shape[0]

    def body(x_ref, mo_ref, w1a_ref, w1m_ref, b1_ref, w2_ref, b2_ref,
             gws_ref, bws_ref, bgs_ref, bbs_ref, *outs):
        mo = mo_ref[...]
        acc = _dot(x_ref[...], w1a_ref[...]) + _dot(mo, w1m_ref[...]) + b1_ref[...]
        y = _silu(_dot(_silu(acc), w2_ref[...]) + b2_ref[...])
        for l in range(nl4):
            outs[l][...] = _dot(y, gws_ref[l]) + bgs_ref[l]
            outs[nl4 + l][...] = _dot(y, bws_ref[l]) + bbs_ref[l]

    full = lambda s: pl.BlockSpec(s, lambda: tuple(0 for _ in s))
    args = (x256, moments, w1a, w1m, b1, w2, b2, gws, bws, bgs, bbs)
    outs = pl.pallas_call(
        body,
        in_specs=[full(a.shape) for a in args],
        out_specs=[full((bsz, 64))] * (2 * nl4),
        out_shape=[jax.ShapeDtypeStruct((bsz, 64), F32)] * (2 * nl4),
    )(*args)
    return outs[:nl4], outs[nl4:]


def _ada_ln_block(h, g_exp, be_exp):
    mu = jnp.mean(h, axis=1, keepdims=True)
    var = jnp.mean((h - mu) ** 2, axis=1, keepdims=True)
    return ((h - mu) / jnp.sqrt(var + 1e-6)) * (1.0 + g_exp) + be_exp


def _onehot_mol(nper):
    def f(pid, bsz):
        r = lax.broadcasted_iota(jnp.int32, (NBLK, bsz), 0)
        c = lax.broadcasted_iota(jnp.int32, (NBLK, bsz), 1)
        mol = pid * (NBLK // nper) + r // nper
        return (mol == c).astype(F32)
    return f


def _pre(h, ga, ba, coords, wcat_s, wcat_d, p3, nper):
    n = h.shape[0]
    bsz = ga.shape[0]
    onehot = _onehot_mol(nper)

    def body(h_ref, ga_ref, ba_ref, c_ref, ws_ref, wd_ref, p3_ref,
             hn_ref, ts_ref, td_ref):
        pid = pl.program_id(0)
        oh = onehot(pid, bsz)
        g_exp = _dot(oh, ga_ref[...])
        be_exp = _dot(oh, ba_ref[...])
        hn = _ada_ln_block(h_ref[...], g_exp, be_exp)
        hn_ref[...] = hn
        cpart = _dot(c_ref[...], p3_ref[...])
        ts_ref[...] = _dot(hn, ws_ref[...]) + cpart
        td_ref[...] = _dot(hn, wd_ref[...]) + cpart

    full = lambda s: pl.BlockSpec(s, lambda i: tuple(0 for _ in s))
    return pl.pallas_call(
        body,
        grid=(n // NBLK,),
        in_specs=[
            pl.BlockSpec((NBLK, 64), lambda i: (i, 0)),
            full(ga.shape), full(ba.shape),
            pl.BlockSpec((NBLK, 3), lambda i: (i, 0)),
            full(wcat_s.shape), full(wcat_d.shape), full((3, 128)),
        ],
        out_specs=[
            pl.BlockSpec((NBLK, 64), lambda i: (i, 0)),
            pl.BlockSpec((NBLK, 128), lambda i: (i, 0)),
            pl.BlockSpec((NBLK, 128), lambda i: (i, 0)),
        ],
        out_shape=[
            jax.ShapeDtypeStruct((n, 64), F32),
            jax.ShapeDtypeStruct((n, 128), F32),
            jax.ShapeDtypeStruct((n, 128), F32),
        ],
    )(h, ga, ba, coords, wcat_s, wcat_d, p3)


def _edge(ss, sd, a, consts, layer0, out_a):
    e = ss.shape[0]
    grid = (e // EBLK,)

    def body(*refs):
        i = 0
        ss_ref = refs[i]; i += 1
        sd_ref = refs[i]; i += 1
        if layer0:
            r64_ref = refs[i]; i += 1
            w1p_ref = refs[i]; i += 1
            b1p_ref = refs[i]; i += 1
            w2p_ref = refs[i]; i += 1
            b2p_ref = refs[i]; i += 1
        else:
            a_ref = refs[i]; i += 1
        wm3_ref = refs[i]; i += 1
        wg3_ref = refs[i]; i += 1
        bm_ref = refs[i]; i += 1
        bg_ref = refs[i]; i += 1
        if out_a:
            wa_ref = refs[i]; i += 1
            ba_ref = refs[i]; i += 1
        m_ref = refs[i]; i += 1
        if out_a:
            anew_ref = refs[i]; i += 1

        svs = ss_ref[...]
        svd = sd_ref[...]
        s = svs + svd
        if layer0:
            d = jnp.abs(svs[:, 72:75] - svd[:, 72:75])
            av = _dot(_silu(_dot(d, w1p_ref[...]) + b1p_ref[...]),
                      w2p_ref[...]) + b2p_ref[...]
            pgbase = _dot(s[:, 64:72], r64_ref[...])
        else:
            av = a_ref[...]
            pgbase = s[:, 64:128]
        pm = s[:, 0:64] + _dot(av, wm3_ref[...]) + bm_ref[...]
        pg = pgbase + _dot(av, wg3_ref[...]) + bg_ref[...]
        m = _silu(pm) * jax.nn.sigmoid(pg)
        m_ref[...] = jnp.concatenate([m, jnp.zeros_like(m)], axis=1)
        if out_a:
            anew_ref[...] = av + _dot(m, wa_ref[...]) + ba_ref[...]

    eb = lambda w: pl.BlockSpec((EBLK, w), lambda i: (i, 0))
    full = lambda s: pl.BlockSpec(s, lambda i: tuple(0 for _ in s))

    ins = [ss, sd]
    in_specs = [eb(128), eb(128)]
    if layer0:
        for k in ('r64', 'w1p', 'b1p', 'w2p', 'b2p'):
            ins.append(consts[k]); in_specs.append(full(consts[k].shape))
    else:
        ins.append(a); in_specs.append(eb(16))
    for k in ('wm3', 'wg3', 'bm', 'bg'):
        ins.append(consts[k]); in_specs.append(full(consts[k].shape))
    if out_a:
        for k in ('wa', 'ba'):
            ins.append(consts[k]); in_specs.append(full(consts[k].shape))

    out_specs = [eb(128)]
    out_shape = [jax.ShapeDtypeStruct((e, 128), F32)]
    if out_a:
        out_specs.append(eb(16))
        out_shape.append(jax.ShapeDtypeStruct((e, 16), F32))
    outs = pl.pallas_call(
        body,
        grid=grid,
        in_specs=in_specs,
        out_specs=out_specs,
        out_shape=out_shape,
    )(*ins)
    return outs if out_a else (outs[0], None)


def _node_update(h, hn, agg2, wu1h, wu1a, bu1, wu2, bu2):
    n = h.shape[0]

    def body(h_ref, hn_ref, ag_ref, w1h_ref, w1a_ref, b1_ref, w2_ref, b2_ref,
             o_ref):
        agg = ag_ref[0] + ag_ref[1]
        u = _silu(_dot(hn_ref[...], w1h_ref[...]) + _dot(agg, w1a_ref[...])
                  + b1_ref[...])
        o_ref[...] = h_ref[...] + _dot(u, w2_ref[...]) + b2_ref[...]

    full = lambda s: pl.BlockSpec(s, lambda i: tuple(0 for _ in s))
    return pl.pallas_call(
        body,
        grid=(n // NBLK,),
        in_specs=[
            pl.BlockSpec((NBLK, 64), lambda i: (i, 0)),
            pl.BlockSpec((NBLK, 64), lambda i: (i, 0)),
            pl.BlockSpec((2, NBLK, 128), lambda i: (0, i, 0)),
            full(wu1h.shape), full(wu1a.shape), full(bu1.shape),
            full(wu2.shape), full(bu2.shape),
        ],
        out_specs=pl.BlockSpec((NBLK, 64), lambda i: (i, 0)),
        out_shape=jax.ShapeDtypeStruct((n, 64), F32),
    )(h, hn, agg2, wu1h, wu1a, bu1, wu2, bu2)


def _head(h, ga, ba, coords, w1, b1, w2, b2, nper):
    n = h.shape[0]
    bsz = ga.shape[0]
    onehot = _onehot_mol(nper)

    def body(h_ref, ga_ref, ba_ref, c_ref, w1_ref, b1_ref, w2_ref, b2_ref,
             o_ref):
        pid = pl.program_id(0)
        oh = onehot(pid, bsz)
        hn = _ada_ln_block(h_ref[...], _dot(oh, ga_ref[...]),
                           _dot(oh, ba_ref[...]))
        u = _silu(_dot(hn, w1_ref[...]) + b1_ref[...])
        o_ref[...] = jnp.sign(c_ref[...]) * (_dot(u, w2_ref[...]) + b2_ref[...])

    full = lambda s: pl.BlockSpec(s, lambda i: tuple(0 for _ in s))
    return pl.pallas_call(
        body,
        grid=(n // NBLK,),
        in_specs=[
            pl.BlockSpec((NBLK, 64), lambda i: (i, 0)),
            full(ga.shape), full(ba.shape),
            pl.BlockSpec((NBLK, 3), lambda i: (i, 0)),
            full(w1.shape), full(b1.shape), full(w2.shape), full(b2.shape),
        ],
        out_specs=pl.BlockSpec((NBLK, 3), lambda i: (i, 0)),
        out_shape=jax.ShapeDtypeStruct((n, 3), F32),
    )(h, ga, ba, coords, w1, b1, w2, b2)


def _sc_gather_pair(ts, td, src, dst):
    e = src.shape[0]
    w = ts.shape[1]
    n = ts.shape[0]
    ew = e // 16
    nfull = ew // CH
    tail = ew - nfull * CH
    stg = (n // 16) // 8 * 8
    stail = n - 16 * stg
    mesh = plsc.VectorSubcoreMesh(core_axis_name="c", subcore_axis_name="s")

    npairs = nfull // 2

    @functools.partial(
        pl.kernel,
        out_type=[jax.ShapeDtypeStruct((e, w), F32)] * 2,
        mesh=mesh,
        scratch_types=[
            pltpu.VMEM_SHARED((n, w), F32),
            [pltpu.VMEM((CH,), jnp.int32) for _ in range(2)],
            [pltpu.VMEM((CH, w), F32) for _ in range(2)],
            [pltpu.SemaphoreType.DMA for _ in range(2)],
            [pltpu.SemaphoreType.DMA for _ in range(2)],
            pltpu.VMEM((tail,), jnp.int32), pltpu.VMEM((tail, w), F32),
        ],
    )
    def run(ts_h, td_h, src_h, dst_h, os_h, od_h, tspm, ix, rows, semi, semw,
            ix_t, rows_t):
        cid = lax.axis_index("c")
        sid = lax.axis_index("s")

        def stage(tab_h):
            pltpu.sync_copy(tab_h.at[pl.ds(sid * stg, stg)],
                            tspm.at[pl.ds(sid * stg, stg)])
            if stail:
                @pl.when(sid == 0)
                def _():
                    pltpu.sync_copy(tab_h.at[pl.ds(16 * stg, stail)],
                                    tspm.at[pl.ds(16 * stg, stail)])

        def side(idx_h, out_h):
            base = sid * ew

            def issue_idx(j, b):
                pltpu.async_copy(idx_h.at[pl.ds(base + j * CH, CH)], ix[b],
                                 semi[b])

            issue_idx(0, 0)
            issue_idx(1, 1)

            @pl.loop(0, npairs)
            def _(jj):
                for b in range(2):
                    j = 2 * jj + b
                    pltpu.make_async_copy(idx_h.at[pl.ds(base, CH)], ix[b],
                                          semi[b]).wait()

                    @pl.when(jj >= 1)
                    def _():
                        pltpu.make_async_copy(rows[b],
                                              out_h.at[pl.ds(base, CH)],
                                              semw[b]).wait()
                    pltpu.sync_copy(tspm.at[ix[b]], rows[b])
                    pltpu.async_copy(rows[b], out_h.at[pl.ds(j * CH + base, CH)],
                                     semw[b])

                    @pl.when(jj + 1 < npairs)
                    def _():
                        issue_idx(j + 2, b)

            for b in range(2):
                pltpu.make_async_copy(rows[b], out_h.at[pl.ds(base, CH)],
                                      semw[b]).wait()

            if tail:
                off = base + nfull * CH
                pltpu.sync_copy(idx_h.at[pl.ds(off, tail)], ix_t)
                pltpu.sync_copy(tspm.at[ix_t], rows_t)
                pltpu.sync_copy(rows_t, out_h.at[pl.ds(off, tail)])

        @pl.when(cid == 0)
        def _():
            stage(ts_h)

        @pl.when(cid == 1)
        def _():
            stage(td_h)

        plsc.subcore_barrier()

        @pl.when(cid == 0)
        def _():
            side(src_h, os_h)

        @pl.when(cid == 1)
        def _():
            side(dst_h, od_h)

    return run(ts, td, src, dst)


def _sc_scatter(m, dst, zeros):
    e = m.shape[0]
    w = m.shape[1]
    rows_per_tile = zeros.shape[0]
    n = rows_per_tile * 16
    ew = e // NW
    nfull = ew // CH
    tail = ew - nfull * CH
    mesh = plsc.VectorSubcoreMesh(core_axis_name="c", subcore_axis_name="s")

    npairs = nfull // 2

    @functools.partial(
        pl.kernel,
        out_type=jax.ShapeDtypeStruct((2 * n, w), F32),
        mesh=mesh,
        scratch_types=[
            pltpu.VMEM_SHARED((n, w), F32),
            [pltpu.VMEM((CH, w), F32) for _ in range(2)],
            [pltpu.VMEM((CH,), jnp.int32) for _ in range(2)],
            [pltpu.SemaphoreType.DMA for _ in range(2)],
            [pltpu.SemaphoreType.DMA for _ in range(2)],
            pltpu.VMEM((tail, w), F32), pltpu.VMEM((tail,), jnp.int32),
        ],
    )
    def run(m_h, dst_h, z_h, out_h, acc, mr, ix, semm, semix, mr_t, ix_t):
        cid = lax.axis_index("c")
        sid = lax.axis_index("s")
        wid = cid * 16 + sid
        base = wid * ew
        pltpu.sync_copy(z_h, acc.at[pl.ds(sid * rows_per_tile, rows_per_tile)])
        plsc.subcore_barrier()

        def issue(j, b):
            off = base + j * CH
            pltpu.async_copy(m_h.at[pl.ds(off, CH)], mr[b], semm[b])
            pltpu.async_copy(dst_h.at[pl.ds(off, CH)], ix[b], semix[b])

        issue(0, 0)
        issue(1, 1)

        @pl.loop(0, npairs)
        def _(jj):
            for b in range(2):
                j = 2 * jj + b
                pltpu.make_async_copy(m_h.at[pl.ds(base, CH)], mr[b],
                                      semm[b]).wait()
                pltpu.make_async_copy(dst_h.at[pl.ds(base, CH)], ix[b],
                                      semix[b]).wait()
                pltpu.sync_copy(mr[b], acc.at[ix[b]], add=True)

                @pl.when(jj + 1 < npairs)
                def _():
                    issue(j + 2, b)

        off = base + nfull * CH
        pltpu.sync_copy(m_h.at[pl.ds(off, tail)], mr_t)
        pltpu.sync_copy(dst_h.at[pl.ds(off, tail)], ix_t)
        pltpu.sync_copy(mr_t, acc.at[ix_t], add=True)

        plsc.subcore_barrier()
        pltpu.sync_copy(
            acc.at[pl.ds(sid * rows_per_tile, rows_per_tile)],
            out_h.at[pl.ds(cid * n + sid * rows_per_tile, rows_per_tile)])

    return run(m, dst, zeros)


def _place(shape, blocks):
    out = jnp.zeros(shape, F32)
    for (r, c, blk) in blocks:
        out = lax.dynamic_update_slice(out, blk.astype(F32), (r, c))
    return out


def kernel(coords, atoms, masses, edge_index, batch_ptrs, moments, t, params):
    p = params
    n = coords.shape[0]
    e = edge_index.shape[1]
    bsz = moments.shape[0]
    nper = n // bsz
    nl = len(p['blocks'])

    src = edge_index[0].astype(jnp.int32)
    dst = edge_index[1].astype(jnp.int32)
    atoms2 = atoms.reshape(n, 1).astype(jnp.int32)
    n_pad = ((n // 16 + 7) // 8 * 8) * 16
    zeros = jnp.zeros((n_pad // 16, 128), F32)

    w1, b1, w2, b2 = p['proj_node']
    h = _node_init(coords, atoms2, masses, p['emb_atom'],
                   w1[0:3], w1[3:35], w1[35:36],
                   b1.reshape(1, -1), w2, b2.reshape(1, -1))

    cw1, cb1, cw2, cb2 = p['proj_cond']
    wcs = [bp['Wc'] for bp in p['blocks']] + [p['head_norm'][0]]
    bcs = [bp['bc'] for bp in p['blocks']] + [p['head_norm'][1]]
    gws = jnp.stack([w[:, :64] for w in wcs])
    bws = jnp.stack([w[:, 64:] for w in wcs])
    bgs = jnp.stack([b[:64].reshape(1, 64) for b in bcs])
    bbs = jnp.stack([b[64:].reshape(1, 64) for b in bcs])
    def sinus(x, lo, hi):
        waves = jnp.asarray(np.geomspace(lo, hi, 32), F32)
        ang = x[..., None] * (2.0 * np.pi / waves)
        return jnp.concatenate([jnp.sin(ang), jnp.cos(ang)], axis=-1).reshape(
            x.shape[0], -1)

    x256 = jnp.concatenate([sinus(t, 0.001, 1.0), sinus(moments, 1e-4, 1e4)],
                           axis=1)
    ga_l, ba_l = _cond(x256, moments, cw1[0:256], cw1[256:259],
                       cb1.reshape(1, -1), cw2, cb2.reshape(1, -1),
                       gws, bws, bgs, bbs)

    r64 = np.zeros((8, 64), np.float32)
    for hh in range(8):
        r64[hh, hh * 8:(hh + 1) * 8] = 1.0
    r64 = jnp.asarray(r64)
    p3 = np.zeros((3, 128), np.float32)
    p3[0:3, 72:75] = np.eye(3, dtype=np.float32)
    p3 = jnp.asarray(p3)
    p3zero = jnp.zeros((3, 128), F32)

    ew1, eb1, ew2, eb2 = p['proj_edge']
    econsts_base = {
        'r64': r64,
        'w1p': ew1, 'b1p': eb1.reshape(1, 3),
        'w2p': ew2, 'b2p': eb2.reshape(1, 16),
    }

    a = None
    for l in range(nl):
        bp = p['blocks'][l]
        wm, wg = bp['Wm'], bp['Wg']
        if l == 0:
            wcat_s = _place((64, 128), [(0, 0, wm[0:64]), (0, 64, wg[0:64])])
            wcat_d = _place((64, 128),
                            [(0, 0, wm[64:128]), (0, 64, wg[64:128])])
            p3l = p3
        else:
            wcat_s = jnp.concatenate([wm[0:64], _dot(wg[0:64], r64)], axis=1)
            wcat_d = jnp.concatenate([wm[64:128], _dot(wg[64:128], r64)],
                                     axis=1)
            p3l = p3zero

        hn, ts, td = _pre(h, ga_l[l], ba_l[l], coords, wcat_s, wcat_d, p3l,
                          nper)
        ss, sd = _sc_gather_pair(ts, td, src, dst)

        consts = dict(econsts_base)
        consts['wm3'] = wm[128:144]
        consts['wg3'] = _dot(wg[128:144], r64)
        consts['bm'] = bp['bm'].reshape(1, 64)
        consts['bg'] = _dot(bp['bg'].reshape(1, 8), r64)
        out_a = 'Wa' in bp
        if out_a:
            consts['wa'] = bp['Wa']
            consts['ba'] = bp['ba'].reshape(1, 16)

        m, a_next = _edge(ss, sd, a, consts, layer0=(l == 0), out_a=out_a)

        agg2 = _sc_scatter(m, dst, zeros).reshape(2, n_pad, 128)[:, :n, :]

        wu1 = bp['Wu1']
        wu1a = _place((128, 256), [(0, 0, wu1[64:128])])
        h = _node_update(h, hn, agg2, wu1[0:64], wu1a,
                         bp['bu1'].reshape(1, -1), bp['Wu2'],
                         bp['bu2'].reshape(1, -1))
        a = a_next

    hw1, hb1, hw2, hb2 = p['head']
    return _head(h, ga_l[nl], ba_l[nl], coords, hw1, hb1.reshape(1, -1),
                 hw2, hb2.reshape(1, -1), nper)


# speedup vs baseline: 1.8079x; 1.8079x over previous
import functools

import numpy as np
import jax
import jax.numpy as jnp
from jax import lax
from jax.experimental import pallas as pl
from jax.experimental.pallas import tpu as pltpu
from jax.experimental.pallas import tpu_sc as plsc

F32 = jnp.float32
NBLK = 1000
EBLK = 3200
NW = 32
CH = 128


def _silu(x):
    return x * jax.nn.sigmoid(x)


def _dot(a, b):
    return jnp.dot(a, b, preferred_element_type=F32, precision='highest')


def _dot3(a, b):
    return jnp.dot(a, b, preferred_element_type=F32)


def _node_init(coords, atoms2, masses, emb, w1p, w1e, w1m, b1, w2, b2):
    n = coords.shape[0]
    grid = (n // NBLK,)

    def body(c_ref, a_ref, ms_ref, emb_ref, w1p_ref, w1e_ref, w1m_ref,
             b1_ref, w2_ref, b2_ref, o_ref):
        pos = jnp.abs(c_ref[...])
        at = a_ref[...]
        oh = (at == lax.broadcasted_iota(jnp.int32, (NBLK, 90), 1)).astype(F32)
        aemb = _dot(oh, emb_ref[...])
        x = (_dot(pos, w1p_ref[...]) + _dot(aemb, w1e_ref[...])
             + _dot(ms_ref[...], w1m_ref[...]) + b1_ref[...])
        o_ref[...] = _dot(_silu(x), w2_ref[...]) + b2_ref[...]

    full = lambda s: pl.BlockSpec(s, lambda i: tuple(0 for _ in s))
    return pl.pallas_call(
        body,
        grid=grid,
        in_specs=[
            pl.BlockSpec((NBLK, 3), lambda i: (i, 0)),
            pl.BlockSpec((NBLK, 1), lambda i: (i, 0)),
            pl.BlockSpec((NBLK, 1), lambda i: (i, 0)),
            full(emb.shape), full(w1p.shape), full(w1e.shape),
            full(w1m.shape), full(b1.shape), full(w2.shape), full(b2.shape),
        ],
        out_specs=pl.BlockSpec((NBLK, 64), lambda i: (i, 0)),
        out_shape=jax.ShapeDtypeStruct((n, 64), F32),
    )(coords, atoms2, masses, emb, w1p, w1e, w1m, b1, w2, b2)


def _cond(x256, moments, w1a, w1m, b1, w2, b2, gws, bws, bgs, bbs):
    bsz = x256.shape[0]
    nl4 = gws.shape[0]

    def body(x_ref, mo_ref, w1a_ref, w1m_ref, b1_ref, w2_ref, b2_ref,
             gws_ref, bws_ref, bgs_ref, bbs_ref, *outs):
        mo = mo_ref[...]
        acc = _dot(x_ref[...], w1a_ref[...]) + _dot(mo, w1m_ref[...]) + b1_ref[...]
        y = _silu(_dot(_silu(acc), w2_ref[...]) + b2_ref[...])
        for l in range(nl4):
            outs[l][...] = _dot(y, gws_ref[l]) + bgs_ref[l]
            outs[nl4 + l][...] = _dot(y, bws_ref[l]) + bbs_ref[l]

    full = lambda s: pl.BlockSpec(s, lambda: tuple(0 for _ in s))
    args = (x256, moments, w1a, w1m, b1, w2, b2, gws, bws, bgs, bbs)
    outs = pl.pallas_call(
        body,
        in_specs=[full(a.shape) for a in args],
        out_specs=[full((bsz, 64))] * (2 * nl4),
        out_shape=[jax.ShapeDtypeStruct((bsz, 64), F32)] * (2 * nl4),
    )(*args)
    return outs[:nl4], outs[nl4:]


def _ada_ln_block(h, g_exp, be_exp):
    mu = jnp.mean(h, axis=1, keepdims=True)
    var = jnp.mean((h - mu) ** 2, axis=1, keepdims=True)
    return ((h - mu) / jnp.sqrt(var + 1e-6)) * (1.0 + g_exp) + be_exp


def _onehot_mol(nper):
    def f(pid, bsz):
        r = lax.broadcasted_iota(jnp.int32, (NBLK, bsz), 0)
        c = lax.broadcasted_iota(jnp.int32, (NBLK, bsz), 1)
        mol = pid * (NBLK // nper) + r // nper
        return (mol == c).astype(F32)
    return f


def _pre(h, ga, ba, coords, wcat_s, wcat_d, p3, nper):
    n = h.shape[0]
    bsz = ga.shape[0]
    onehot = _onehot_mol(nper)

    def body(h_ref, ga_ref, ba_ref, c_ref, ws_ref, wd_ref, p3_ref,
             hn_ref, ts_ref, td_ref):
        pid = pl.program_id(0)
        oh = onehot(pid, bsz)
        g_exp = _dot(oh, ga_ref[...])
        be_exp = _dot(oh, ba_ref[...])
        hn = _ada_ln_block(h_ref[...], g_exp, be_exp)
        hn_ref[...] = hn
        cpart = _dot(c_ref[...], p3_ref[...])
        ts_ref[...] = _dot(hn, ws_ref[...]) + cpart
        td_ref[...] = _dot(hn, wd_ref[...]) + cpart

    full = lambda s: pl.BlockSpec(s, lambda i: tuple(0 for _ in s))
    return pl.pallas_call(
        body,
        grid=(n // NBLK,),
        in_specs=[
            pl.BlockSpec((NBLK, 64), lambda i: (i, 0)),
            full(ga.shape), full(ba.shape),
            pl.BlockSpec((NBLK, 3), lambda i: (i, 0)),
            full(wcat_s.shape), full(wcat_d.shape), full((3, 128)),
        ],
        out_specs=[
            pl.BlockSpec((NBLK, 64), lambda i: (i, 0)),
            pl.BlockSpec((NBLK, 128), lambda i: (i, 0)),
            pl.BlockSpec((NBLK, 128), lambda i: (i, 0)),
        ],
        out_shape=[
            jax.ShapeDtypeStruct((n, 64), F32),
            jax.ShapeDtypeStruct((n, 128), F32),
            jax.ShapeDtypeStruct((n, 128), F32),
        ],
    )(h, ga, ba, coords, wcat_s, wcat_d, p3)


def _edge(ss, sd, a, consts, layer0, out_a):
    e = ss.shape[0]
    grid = (e // EBLK,)

    def body(*refs):
        i = 0
        ss_ref = refs[i]; i += 1
        sd_ref = refs[i]; i += 1
        if layer0:
            r64_ref = refs[i]; i += 1
            w1p_ref = refs[i]; i += 1
            b1p_ref = refs[i]; i += 1
            w2p_ref = refs[i]; i += 1
            b2p_ref = refs[i]; i += 1
        else:
            a_ref = refs[i]; i += 1
        wm3_ref = refs[i]; i += 1
        wg3_ref = refs[i]; i += 1
        bm_ref = refs[i]; i += 1
        bg_ref = refs[i]; i += 1
        if out_a:
            wa_ref = refs[i]; i += 1
            ba_ref = refs[i]; i += 1
        m_ref = refs[i]; i += 1
        if out_a:
            anew_ref = refs[i]; i += 1

        svs = ss_ref[...]
        svd = sd_ref[...]
        s = svs + svd
        if layer0:
            d = jnp.abs(svs[:, 72:75] - svd[:, 72:75])
            av = _dot3(_silu(_dot3(d, w1p_ref[...]) + b1p_ref[...]),
                      w2p_ref[...]) + b2p_ref[...]
            pgbase = _dot3(s[:, 64:72], r64_ref[...])
        else:
            av = a_ref[...]
            pgbase = s[:, 64:128]
        pm = s[:, 0:64] + _dot3(av, wm3_ref[...]) + bm_ref[...]
        pg = pgbase + _dot3(av, wg3_ref[...]) + bg_ref[...]
        m = _silu(pm) * jax.nn.sigmoid(pg)
        m_ref[...] = jnp.concatenate([m, jnp.zeros_like(m)], axis=1)
        if out_a:
            anew_ref[...] = av + _dot3(m, wa_ref[...]) + ba_ref[...]

    eb = lambda w: pl.BlockSpec((EBLK, w), lambda i: (i, 0))
    full = lambda s: pl.BlockSpec(s, lambda i: tuple(0 for _ in s))

    ins = [ss, sd]
    in_specs = [eb(128), eb(128)]
    if layer0:
        for k in ('r64', 'w1p', 'b1p', 'w2p', 'b2p'):
            ins.append(consts[k]); in_specs.append(full(consts[k].shape))
    else:
        ins.append(a); in_specs.append(eb(16))
    for k in ('wm3', 'wg3', 'bm', 'bg'):
        ins.append(consts[k]); in_specs.append(full(consts[k].shape))
    if out_a:
        for k in ('wa', 'ba'):
            ins.append(consts[k]); in_specs.append(full(consts[k].shape))

    out_specs = [eb(128)]
    out_shape = [jax.ShapeDtypeStruct((e, 128), F32)]
    if out_a:
        out_specs.append(eb(16))
        out_shape.append(jax.ShapeDtypeStruct((e, 16), F32))
    outs = pl.pallas_call(
        body,
        grid=grid,
        in_specs=in_specs,
        out_specs=out_specs,
        out_shape=out_shape,
    )(*ins)
    return outs if out_a else (outs[0], None)


def _node_update(h, hn, agg2, wu1h, wu1a, bu1, wu2, bu2):
    n = h.shape[0]

    def body(h_ref, hn_ref, ag_ref, w1h_ref, w1a_ref, b1_ref, w2_ref, b2_ref,
             o_ref):
        agg = ag_ref[0] + ag_ref[1]
        u = _silu(_dot(hn_ref[...], w1h_ref[...]) + _dot(agg, w1a_ref[...])
                  + b1_ref[...])
        o_ref[...] = h_ref[...] + _dot(u, w2_ref[...]) + b2_ref[...]

    full = lambda s: pl.BlockSpec(s, lambda i: tuple(0 for _ in s))
    return pl.pallas_call(
        body,
        grid=(n // NBLK,),
        in_specs=[
            pl.BlockSpec((NBLK, 64), lambda i: (i, 0)),
            pl.BlockSpec((NBLK, 64), lambda i: (i, 0)),
            pl.BlockSpec((2, NBLK, 128), lambda i: (0, i, 0)),
            full(wu1h.shape), full(wu1a.shape), full(bu1.shape),
            full(wu2.shape), full(bu2.shape),
        ],
        out_specs=pl.BlockSpec((NBLK, 64), lambda i: (i, 0)),
        out_shape=jax.ShapeDtypeStruct((n, 64), F32),
    )(h, hn, agg2, wu1h, wu1a, bu1, wu2, bu2)


def _head(h, ga, ba, coords, w1, b1, w2, b2, nper):
    n = h.shape[0]
    bsz = ga.shape[0]
    onehot = _onehot_mol(nper)

    def body(h_ref, ga_ref, ba_ref, c_ref, w1_ref, b1_ref, w2_ref, b2_ref,
             o_ref):
        pid = pl.program_id(0)
        oh = onehot(pid, bsz)
        hn = _ada_ln_block(h_ref[...], _dot(oh, ga_ref[...]),
                           _dot(oh, ba_ref[...]))
        u = _silu(_dot(hn, w1_ref[...]) + b1_ref[...])
        o_ref[...] = jnp.sign(c_ref[...]) * (_dot(u, w2_ref[...]) + b2_ref[...])

    full = lambda s: pl.BlockSpec(s, lambda i: tuple(0 for _ in s))
    return pl.pallas_call(
        body,
        grid=(n // NBLK,),
        in_specs=[
            pl.BlockSpec((NBLK, 64), lambda i: (i, 0)),
            full(ga.shape), full(ba.shape),
            pl.BlockSpec((NBLK, 3), lambda i: (i, 0)),
            full(w1.shape), full(b1.shape), full(w2.shape), full(b2.shape),
        ],
        out_specs=pl.BlockSpec((NBLK, 3), lambda i: (i, 0)),
        out_shape=jax.ShapeDtypeStruct((n, 3), F32),
    )(h, ga, ba, coords, w1, b1, w2, b2)


def _sc_gather_pair(ts, td, src, dst):
    e = src.shape[0]
    w = ts.shape[1]
    n = ts.shape[0]
    ew = e // 16
    nfull = ew // CH
    tail = ew - nfull * CH
    stg = (n // 16) // 8 * 8
    stail = n - 16 * stg
    mesh = plsc.VectorSubcoreMesh(core_axis_name="c", subcore_axis_name="s")

    npairs = nfull // 2

    @functools.partial(
        pl.kernel,
        out_type=[jax.ShapeDtypeStruct((e, w), F32)] * 2,
        mesh=mesh,
        scratch_types=[
            pltpu.VMEM_SHARED((n, w), F32),
            [pltpu.VMEM((CH,), jnp.int32) for _ in range(2)],
            [pltpu.VMEM((CH, w), F32) for _ in range(2)],
            [pltpu.SemaphoreType.DMA for _ in range(2)],
            [pltpu.SemaphoreType.DMA for _ in range(2)],
            pltpu.VMEM((tail,), jnp.int32), pltpu.VMEM((tail, w), F32),
        ],
    )
    def run(ts_h, td_h, src_h, dst_h, os_h, od_h, tspm, ix, rows, semi, semw,
            ix_t, rows_t):
        cid = lax.axis_index("c")
        sid = lax.axis_index("s")

        def stage(tab_h):
            pltpu.sync_copy(tab_h.at[pl.ds(sid * stg, stg)],
                            tspm.at[pl.ds(sid * stg, stg)])
            if stail:
                @pl.when(sid == 0)
                def _():
                    pltpu.sync_copy(tab_h.at[pl.ds(16 * stg, stail)],
                                    tspm.at[pl.ds(16 * stg, stail)])

        def side(idx_h, out_h):
            base = sid * ew

            def issue_idx(j, b):
                pltpu.async_copy(idx_h.at[pl.ds(base + j * CH, CH)], ix[b],
                                 semi[b])

            issue_idx(0, 0)
            issue_idx(1, 1)

            @pl.loop(0, npairs)
            def _(jj):
                for b in range(2):
                    j = 2 * jj + b
                    pltpu.make_async_copy(idx_h.at[pl.ds(base, CH)], ix[b],
                                          semi[b]).wait()

                    @pl.when(jj >= 1)
                    def _():
                        pltpu.make_async_copy(rows[b],
                                              out_h.at[pl.ds(base, CH)],
                                              semw[b]).wait()
                    pltpu.sync_copy(tspm.at[ix[b]], rows[b])
                    pltpu.async_copy(rows[b], out_h.at[pl.ds(j * CH + base, CH)],
                                     semw[b])

                    @pl.when(jj + 1 < npairs)
                    def _():
                        issue_idx(j + 2, b)

            for b in range(2):
                pltpu.make_async_copy(rows[b], out_h.at[pl.ds(base, CH)],
                                      semw[b]).wait()

            if tail:
                off = base + nfull * CH
                pltpu.sync_copy(idx_h.at[pl.ds(off, tail)], ix_t)
                pltpu.sync_copy(tspm.at[ix_t], rows_t)
                pltpu.sync_copy(rows_t, out_h.at[pl.ds(off, tail)])

        @pl.when(cid == 0)
        def _():
            stage(ts_h)

        @pl.when(cid == 1)
        def _():
            stage(td_h)

        plsc.subcore_barrier()

        @pl.when(cid == 0)
        def _():
            side(src_h, os_h)

        @pl.when(cid == 1)
        def _():
            side(dst_h, od_h)

    return run(ts, td, src, dst)


def _sc_scatter(m, dst, zeros):
    e = m.shape[0]
    w = m.shape[1]
    rows_per_tile = zeros.shape[0]
    n = rows_per_tile * 16
    ew = e // NW
    nfull = ew // CH
    tail = ew - nfull * CH
    mesh = plsc.VectorSubcoreMesh(core_axis_name="c", subcore_axis_name="s")

    npairs = nfull // 2

    @functools.partial(
        pl.kernel,
        out_type=jax.ShapeDtypeStruct((2 * n, w), F32),
        mesh=mesh,
        scratch_types=[
            pltpu.VMEM_SHARED((n, w), F32),
            [pltpu.VMEM((CH, w), F32) for _ in range(2)],
            [pltpu.VMEM((CH,), jnp.int32) for _ in range(2)],
            [pltpu.SemaphoreType.DMA for _ in range(2)],
            [pltpu.SemaphoreType.DMA for _ in range(2)],
            pltpu.VMEM((tail, w), F32), pltpu.VMEM((tail,), jnp.int32),
        ],
    )
    def run(m_h, dst_h, z_h, out_h, acc, mr, ix, semm, semix, mr_t, ix_t):
        cid = lax.axis_index("c")
        sid = lax.axis_index("s")
        wid = cid * 16 + sid
        base = wid * ew
        pltpu.sync_copy(z_h, acc.at[pl.ds(sid * rows_per_tile, rows_per_tile)])
        plsc.subcore_barrier()

        def issue(j, b):
            off = base + j * CH
            pltpu.async_copy(m_h.at[pl.ds(off, CH)], mr[b], semm[b])
            pltpu.async_copy(dst_h.at[pl.ds(off, CH)], ix[b], semix[b])

        issue(0, 0)
        issue(1, 1)

        @pl.loop(0, npairs)
        def _(jj):
            for b in range(2):
                j = 2 * jj + b
                pltpu.make_async_copy(m_h.at[pl.ds(base, CH)], mr[b],
                                      semm[b]).wait()
                pltpu.make_async_copy(dst_h.at[pl.ds(base, CH)], ix[b],
                                      semix[b]).wait()
                pltpu.sync_copy(mr[b], acc.at[ix[b]], add=True)

                @pl.when(jj + 1 < npairs)
                def _():
                    issue(j + 2, b)

        off = base + nfull * CH
        pltpu.sync_copy(m_h.at[pl.ds(off, tail)], mr_t)
        pltpu.sync_copy(dst_h.at[pl.ds(off, tail)], ix_t)
        pltpu.sync_copy(mr_t, acc.at[ix_t], add=True)

        plsc.subcore_barrier()
        pltpu.sync_copy(
            acc.at[pl.ds(sid * rows_per_tile, rows_per_tile)],
            out_h.at[pl.ds(cid * n + sid * rows_per_tile, rows_per_tile)])

    return run(m, dst, zeros)


def _place(shape, blocks):
    out = jnp.zeros(shape, F32)
    for (r, c, blk) in blocks:
        out = lax.dynamic_update_slice(out, blk.astype(F32), (r, c))
    return out


def kernel(coords, atoms, masses, edge_index, batch_ptrs, moments, t, params):
    p = params
    n = coords.shape[0]
    e = edge_index.shape[1]
    bsz = moments.shape[0]
    nper = n // bsz
    nl = len(p['blocks'])

    src = edge_index[0].astype(jnp.int32)
    dst = edge_index[1].astype(jnp.int32)
    atoms2 = atoms.reshape(n, 1).astype(jnp.int32)
    n_pad = ((n // 16 + 7) // 8 * 8) * 16
    zeros = jnp.zeros((n_pad // 16, 128), F32)

    w1, b1, w2, b2 = p['proj_node']
    h = _node_init(coords, atoms2, masses, p['emb_atom'],
                   w1[0:3], w1[3:35], w1[35:36],
                   b1.reshape(1, -1), w2, b2.reshape(1, -1))

    cw1, cb1, cw2, cb2 = p['proj_cond']
    wcs = [bp['Wc'] for bp in p['blocks']] + [p['head_norm'][0]]
    bcs = [bp['bc'] for bp in p['blocks']] + [p['head_norm'][1]]
    gws = jnp.stack([w[:, :64] for w in wcs])
    bws = jnp.stack([w[:, 64:] for w in wcs])
    bgs = jnp.stack([b[:64].reshape(1, 64) for b in bcs])
    bbs = jnp.stack([b[64:].reshape(1, 64) for b in bcs])
    def sinus(x, lo, hi):
        waves = jnp.asarray(np.geomspace(lo, hi, 32), F32)
        ang = x[..., None] * (2.0 * np.pi / waves)
        return jnp.concatenate([jnp.sin(ang), jnp.cos(ang)], axis=-1).reshape(
            x.shape[0], -1)

    x256 = jnp.concatenate([sinus(t, 0.001, 1.0), sinus(moments, 1e-4, 1e4)],
                           axis=1)
    ga_l, ba_l = _cond(x256, moments, cw1[0:256], cw1[256:259],
                       cb1.reshape(1, -1), cw2, cb2.reshape(1, -1),
                       gws, bws, bgs, bbs)

    r64 = np.zeros((8, 64), np.float32)
    for hh in range(8):
        r64[hh, hh * 8:(hh + 1) * 8] = 1.0
    r64 = jnp.asarray(r64)
    p3 = np.zeros((3, 128), np.float32)
    p3[0:3, 72:75] = np.eye(3, dtype=np.float32)
    p3 = jnp.asarray(p3)
    p3zero = jnp.zeros((3, 128), F32)

    ew1, eb1, ew2, eb2 = p['proj_edge']
    econsts_base = {
        'r64': r64,
        'w1p': ew1, 'b1p': eb1.reshape(1, 3),
        'w2p': ew2, 'b2p': eb2.reshape(1, 16),
    }

    a = None
    for l in range(nl):
        bp = p['blocks'][l]
        wm, wg = bp['Wm'], bp['Wg']
        if l == 0:
            wcat_s = _place((64, 128), [(0, 0, wm[0:64]), (0, 64, wg[0:64])])
            wcat_d = _place((64, 128),
                            [(0, 0, wm[64:128]), (0, 64, wg[64:128])])
            p3l = p3
        else:
            wcat_s = jnp.concatenate([wm[0:64], _dot(wg[0:64], r64)], axis=1)
            wcat_d = jnp.concatenate([wm[64:128], _dot(wg[64:128], r64)],
                                     axis=1)
            p3l = p3zero

        hn, ts, td = _pre(h, ga_l[l], ba_l[l], coords, wcat_s, wcat_d, p3l,
                          nper)
        ss, sd = _sc_gather_pair(ts, td, src, dst)

        consts = dict(econsts_base)
        consts['wm3'] = wm[128:144]
        consts['wg3'] = _dot(wg[128:144], r64)
        consts['bm'] = bp['bm'].reshape(1, 64)
        consts['bg'] = _dot(bp['bg'].reshape(1, 8), r64)
        out_a = 'Wa' in bp
        if out_a:
            consts['wa'] = bp['Wa']
            consts['ba'] = bp['ba'].reshape(1, 16)

        m, a_next = _edge(ss, sd, a, consts, layer0=(l == 0), out_a=out_a)

        agg2 = _sc_scatter(m, dst, zeros).reshape(2, n_pad, 128)[:, :n, :]

        wu1 = bp['Wu1']
        wu1a = _place((128, 256), [(0, 0, wu1[64:128])])
        h = _node_update(h, hn, agg2, wu1[0:64], wu1a,
                         bp['bu1'].reshape(1, -1), bp['Wu2'],
                         bp['bu2'].reshape(1, -1))
        a = a_next

    hw1, hb1, hw2, hb2 = p['head']
    return _head(h, ga_l[nl], ba_l[nl], coords, hw1, hb1.reshape(1, -1),
                 hw2, hb2.reshape(1, -1), nper)


# speedup vs baseline: 1.8678x; 1.0331x over previous
import functools

import numpy as np
import jax
import jax.numpy as jnp
from jax import lax
from jax.experimental import pallas as pl
from jax.experimental.pallas import tpu as pltpu
from jax.experimental.pallas import tpu_sc as plsc

F32 = jnp.float32
NBLK = 1000
EBLK = 6400
NW = 32
CH = 128


def _silu(x):
    return x * jax.nn.sigmoid(x)


def _dot(a, b):
    return jnp.dot(a, b, preferred_element_type=F32, precision='highest')


def _dot3(a, b):
    return jnp.dot(a, b, preferred_element_type=F32)


def _node_init(coords, atoms2, masses, emb, w1p, w1e, w1m, b1, w2, b2):
    n = coords.shape[0]
    grid = (n // NBLK,)

    def body(c_ref, a_ref, ms_ref, emb_ref, w1p_ref, w1e_ref, w1m_ref,
             b1_ref, w2_ref, b2_ref, o_ref):
        pos = jnp.abs(c_ref[...])
        at = a_ref[...]
        oh = (at == lax.broadcasted_iota(jnp.int32, (NBLK, 90), 1)).astype(F32)
        aemb = _dot(oh, emb_ref[...])
        x = (_dot(pos, w1p_ref[...]) + _dot(aemb, w1e_ref[...])
             + _dot(ms_ref[...], w1m_ref[...]) + b1_ref[...])
        o_ref[...] = _dot(_silu(x), w2_ref[...]) + b2_ref[...]

    full = lambda s: pl.BlockSpec(s, lambda i: tuple(0 for _ in s))
    return pl.pallas_call(
        body,
        grid=grid,
        in_specs=[
            pl.BlockSpec((NBLK, 3), lambda i: (i, 0)),
            pl.BlockSpec((NBLK, 1), lambda i: (i, 0)),
            pl.BlockSpec((NBLK, 1), lambda i: (i, 0)),
            full(emb.shape), full(w1p.shape), full(w1e.shape),
            full(w1m.shape), full(b1.shape), full(w2.shape), full(b2.shape),
        ],
        out_specs=pl.BlockSpec((NBLK, 64), lambda i: (i, 0)),
        out_shape=jax.ShapeDtypeStruct((n, 64), F32),
    )(coords, atoms2, masses, emb, w1p, w1e, w1m, b1, w2, b2)


def _cond(x256, moments, w1a, w1m, b1, w2, b2, gws, bws, bgs, bbs):
    bsz = x256.shape[0]
    nl4 = gws.shape[0]

    def body(x_ref, mo_ref, w1a_ref, w1m_ref, b1_ref, w2_ref, b2_ref,
             gws_ref, bws_ref, bgs_ref, bbs_ref, *outs):
        mo = mo_ref[...]
        acc = _dot(x_ref[...], w1a_ref[...]) + _dot(mo, w1m_ref[...]) + b1_ref[...]
        y = _silu(_dot(_silu(acc), w2_ref[...]) + b2_ref[...])
        for l in range(nl4):
            outs[l][...] = _dot(y, gws_ref[l]) + bgs_ref[l]
            outs[nl4 + l][...] = _dot(y, bws_ref[l]) + bbs_ref[l]

    full = lambda s: pl.BlockSpec(s, lambda: tuple(0 for _ in s))
    args = (x256, moments, w1a, w1m, b1, w2, b2, gws, bws, bgs, bbs)
    outs = pl.pallas_call(
        body,
        in_specs=[full(a.shape) for a in args],
        out_specs=[full((bsz, 64))] * (2 * nl4),
        out_shape=[jax.ShapeDtypeStruct((bsz, 64), F32)] * (2 * nl4),
    )(*args)
    return outs[:nl4], outs[nl4:]


def _ada_ln_block(h, g_exp, be_exp):
    mu = jnp.mean(h, axis=1, keepdims=True)
    var = jnp.mean((h - mu) ** 2, axis=1, keepdims=True)
    return ((h - mu) / jnp.sqrt(var + 1e-6)) * (1.0 + g_exp) + be_exp


def _onehot_mol(nper):
    def f(pid, bsz):
        r = lax.broadcasted_iota(jnp.int32, (NBLK, bsz), 0)
        c = lax.broadcasted_iota(jnp.int32, (NBLK, bsz), 1)
        mol = pid * (NBLK // nper) + r // nper
        return (mol == c).astype(F32)
    return f


def _pre(h, ga, ba, coords, wcat_s, wcat_d, p3, nper):
    n = h.shape[0]
    bsz = ga.shape[0]
    onehot = _onehot_mol(nper)

    def body(h_ref, ga_ref, ba_ref, c_ref, ws_ref, wd_ref, p3_ref,
             hn_ref, ts_ref, td_ref):
        pid = pl.program_id(0)
        oh = onehot(pid, bsz)
        g_exp = _dot(oh, ga_ref[...])
        be_exp = _dot(oh, ba_ref[...])
        hn = _ada_ln_block(h_ref[...], g_exp, be_exp)
        hn_ref[...] = hn
        cpart = _dot(c_ref[...], p3_ref[...])
        ts_ref[...] = _dot(hn, ws_ref[...]) + cpart
        td_ref[...] = _dot(hn, wd_ref[...]) + cpart

    full = lambda s: pl.BlockSpec(s, lambda i: tuple(0 for _ in s))
    return pl.pallas_call(
        body,
        grid=(n // NBLK,),
        in_specs=[
            pl.BlockSpec((NBLK, 64), lambda i: (i, 0)),
            full(ga.shape), full(ba.shape),
            pl.BlockSpec((NBLK, 3), lambda i: (i, 0)),
            full(wcat_s.shape), full(wcat_d.shape), full((3, 128)),
        ],
        out_specs=[
            pl.BlockSpec((NBLK, 64), lambda i: (i, 0)),
            pl.BlockSpec((NBLK, 128), lambda i: (i, 0)),
            pl.BlockSpec((NBLK, 128), lambda i: (i, 0)),
        ],
        out_shape=[
            jax.ShapeDtypeStruct((n, 64), F32),
            jax.ShapeDtypeStruct((n, 128), F32),
            jax.ShapeDtypeStruct((n, 128), F32),
        ],
    )(h, ga, ba, coords, wcat_s, wcat_d, p3)


def _edge(ss, sd, a, consts, layer0, out_a):
    e = ss.shape[0]
    grid = (e // EBLK,)

    def body(*refs):
        i = 0
        ss_ref = refs[i]; i += 1
        sd_ref = refs[i]; i += 1
        if layer0:
            r64_ref = refs[i]; i += 1
            w1p_ref = refs[i]; i += 1
            b1p_ref = refs[i]; i += 1
            w2p_ref = refs[i]; i += 1
            b2p_ref = refs[i]; i += 1
        else:
            a_ref = refs[i]; i += 1
        wm3_ref = refs[i]; i += 1
        wg3_ref = refs[i]; i += 1
        bm_ref = refs[i]; i += 1
        bg_ref = refs[i]; i += 1
        if out_a:
            wa_ref = refs[i]; i += 1
            ba_ref = refs[i]; i += 1
        m_ref = refs[i]; i += 1
        if out_a:
            anew_ref = refs[i]; i += 1

        svs = ss_ref[...]
        svd = sd_ref[...]
        s = svs + svd
        if layer0:
            d = jnp.abs(svs[:, 72:75] - svd[:, 72:75])
            av = _dot3(_silu(_dot3(d, w1p_ref[...]) + b1p_ref[...]),
                      w2p_ref[...]) + b2p_ref[...]
            pgbase = _dot3(s[:, 64:72], r64_ref[...])
        else:
            av = a_ref[...]
            pgbase = s[:, 64:128]
        pm = s[:, 0:64] + _dot3(av, wm3_ref[...]) + bm_ref[...]
        pg = pgbase + _dot3(av, wg3_ref[...]) + bg_ref[...]
        m = _silu(pm) * jax.nn.sigmoid(pg)
        m_ref[...] = jnp.concatenate([m, jnp.zeros_like(m)], axis=1)
        if out_a:
            anew_ref[...] = av + _dot3(m, wa_ref[...]) + ba_ref[...]

    eb = lambda w: pl.BlockSpec((EBLK, w), lambda i: (i, 0))
    full = lambda s: pl.BlockSpec(s, lambda i: tuple(0 for _ in s))

    ins = [ss, sd]
    in_specs = [eb(128), eb(128)]
    if layer0:
        for k in ('r64', 'w1p', 'b1p', 'w2p', 'b2p'):
            ins.append(consts[k]); in_specs.append(full(consts[k].shape))
    else:
        ins.append(a); in_specs.append(eb(16))
    for k in ('wm3', 'wg3', 'bm', 'bg'):
        ins.append(consts[k]); in_specs.append(full(consts[k].shape))
    if out_a:
        for k in ('wa', 'ba'):
            ins.append(consts[k]); in_specs.append(full(consts[k].shape))

    out_specs = [eb(128)]
    out_shape = [jax.ShapeDtypeStruct((e, 128), F32)]
    if out_a:
        out_specs.append(eb(16))
        out_shape.append(jax.ShapeDtypeStruct((e, 16), F32))
    outs = pl.pallas_call(
        body,
        grid=grid,
        in_specs=in_specs,
        out_specs=out_specs,
        out_shape=out_shape,
    )(*ins)
    return outs if out_a else (outs[0], None)


def _node_update(h, hn, agg2, wu1h, wu1a, bu1, wu2, bu2):
    n = h.shape[0]

    def body(h_ref, hn_ref, ag_ref, w1h_ref, w1a_ref, b1_ref, w2_ref, b2_ref,
             o_ref):
        agg = ag_ref[0] + ag_ref[1]
        u = _silu(_dot(hn_ref[...], w1h_ref[...]) + _dot(agg, w1a_ref[...])
                  + b1_ref[...])
        o_ref[...] = h_ref[...] + _dot(u, w2_ref[...]) + b2_ref[...]

    full = lambda s: pl.BlockSpec(s, lambda i: tuple(0 for _ in s))
    return pl.pallas_call(
        body,
        grid=(n // NBLK,),
        in_specs=[
            pl.BlockSpec((NBLK, 64), lambda i: (i, 0)),
            pl.BlockSpec((NBLK, 64), lambda i: (i, 0)),
            pl.BlockSpec((2, NBLK, 128), lambda i: (0, i, 0)),
            full(wu1h.shape), full(wu1a.shape), full(bu1.shape),
            full(wu2.shape), full(bu2.shape),
        ],
        out_specs=pl.BlockSpec((NBLK, 64), lambda i: (i, 0)),
        out_shape=jax.ShapeDtypeStruct((n, 64), F32),
    )(h, hn, agg2, wu1h, wu1a, bu1, wu2, bu2)


def _head(h, ga, ba, coords, w1, b1, w2, b2, nper):
    n = h.shape[0]
    bsz = ga.shape[0]
    onehot = _onehot_mol(nper)

    def body(h_ref, ga_ref, ba_ref, c_ref, w1_ref, b1_ref, w2_ref, b2_ref,
             o_ref):
        pid = pl.program_id(0)
        oh = onehot(pid, bsz)
        hn = _ada_ln_block(h_ref[...], _dot(oh, ga_ref[...]),
                           _dot(oh, ba_ref[...]))
        u = _silu(_dot(hn, w1_ref[...]) + b1_ref[...])
        o_ref[...] = jnp.sign(c_ref[...]) * (_dot(u, w2_ref[...]) + b2_ref[...])

    full = lambda s: pl.BlockSpec(s, lambda i: tuple(0 for _ in s))
    return pl.pallas_call(
        body,
        grid=(n // NBLK,),
        in_specs=[
            pl.BlockSpec((NBLK, 64), lambda i: (i, 0)),
            full(ga.shape), full(ba.shape),
            pl.BlockSpec((NBLK, 3), lambda i: (i, 0)),
            full(w1.shape), full(b1.shape), full(w2.shape), full(b2.shape),
        ],
        out_specs=pl.BlockSpec((NBLK, 3), lambda i: (i, 0)),
        out_shape=jax.ShapeDtypeStruct((n, 3), F32),
    )(h, ga, ba, coords, w1, b1, w2, b2)


def _sc_gather_pair(ts, td, src, dst):
    e = src.shape[0]
    w = ts.shape[1]
    n = ts.shape[0]
    ew = e // 16
    nfull = ew // CH
    tail = ew - nfull * CH
    stg = (n // 16) // 8 * 8
    stail = n - 16 * stg
    mesh = plsc.VectorSubcoreMesh(core_axis_name="c", subcore_axis_name="s")

    npairs = nfull // 2

    @functools.partial(
        pl.kernel,
        out_type=[jax.ShapeDtypeStruct((e, w), F32)] * 2,
        mesh=mesh,
        scratch_types=[
            pltpu.VMEM_SHARED((n, w), F32),
            [pltpu.VMEM((CH,), jnp.int32) for _ in range(2)],
            [pltpu.VMEM((CH, w), F32) for _ in range(2)],
            [pltpu.SemaphoreType.DMA for _ in range(2)],
            [pltpu.SemaphoreType.DMA for _ in range(2)],
            pltpu.VMEM((tail,), jnp.int32), pltpu.VMEM((tail, w), F32),
        ],
    )
    def run(ts_h, td_h, src_h, dst_h, os_h, od_h, tspm, ix, rows, semi, semw,
            ix_t, rows_t):
        cid = lax.axis_index("c")
        sid = lax.axis_index("s")

        def stage(tab_h):
            pltpu.sync_copy(tab_h.at[pl.ds(sid * stg, stg)],
                            tspm.at[pl.ds(sid * stg, stg)])
            if stail:
                @pl.when(sid == 0)
                def _():
                    pltpu.sync_copy(tab_h.at[pl.ds(16 * stg, stail)],
                                    tspm.at[pl.ds(16 * stg, stail)])

        def side(idx_h, out_h):
            base = sid * ew

            def issue_idx(j, b):
                pltpu.async_copy(idx_h.at[pl.ds(base + j * CH, CH)], ix[b],
                                 semi[b])

            issue_idx(0, 0)
            issue_idx(1, 1)

            @pl.loop(0, npairs)
            def _(jj):
                for b in range(2):
                    j = 2 * jj + b
                    pltpu.make_async_copy(idx_h.at[pl.ds(base, CH)], ix[b],
                                          semi[b]).wait()

                    @pl.when(jj >= 1)
                    def _():
                        pltpu.make_async_copy(rows[b],
                                              out_h.at[pl.ds(base, CH)],
                                              semw[b]).wait()
                    pltpu.sync_copy(tspm.at[ix[b]], rows[b])
                    pltpu.async_copy(rows[b], out_h.at[pl.ds(j * CH + base, CH)],
                                     semw[b])

                    @pl.when(jj + 1 < npairs)
                    def _():
                        issue_idx(j + 2, b)

            for b in range(2):
                pltpu.make_async_copy(rows[b], out_h.at[pl.ds(base, CH)],
                                      semw[b]).wait()

            if tail:
                off = base + nfull * CH
                pltpu.sync_copy(idx_h.at[pl.ds(off, tail)], ix_t)
                pltpu.sync_copy(tspm.at[ix_t], rows_t)
                pltpu.sync_copy(rows_t, out_h.at[pl.ds(off, tail)])

        @pl.when(cid == 0)
        def _():
            stage(ts_h)

        @pl.when(cid == 1)
        def _():
            stage(td_h)

        plsc.subcore_barrier()

        @pl.when(cid == 0)
        def _():
            side(src_h, os_h)

        @pl.when(cid == 1)
        def _():
            side(dst_h, od_h)

    return run(ts, td, src, dst)


def _sc_scatter(m, dst, zeros):
    e = m.shape[0]
    w = m.shape[1]
    rows_per_tile = zeros.shape[0]
    n = rows_per_tile * 16
    ew = e // NW
    nfull = ew // CH
    tail = ew - nfull * CH
    mesh = plsc.VectorSubcoreMesh(core_axis_name="c", subcore_axis_name="s")

    npairs = nfull // 2

    @functools.partial(
        pl.kernel,
        out_type=jax.ShapeDtypeStruct((2 * n, w), F32),
        mesh=mesh,
        scratch_types=[
            pltpu.VMEM_SHARED((n, w), F32),
            [pltpu.VMEM((CH, w), F32) for _ in range(2)],
            [pltpu.VMEM((CH,), jnp.int32) for _ in range(2)],
            [pltpu.SemaphoreType.DMA for _ in range(2)],
            [pltpu.SemaphoreType.DMA for _ in range(2)],
            pltpu.VMEM((tail, w), F32), pltpu.VMEM((tail,), jnp.int32),
        ],
    )
    def run(m_h, dst_h, z_h, out_h, acc, mr, ix, semm, semix, mr_t, ix_t):
        cid = lax.axis_index("c")
        sid = lax.axis_index("s")
        wid = cid * 16 + sid
        base = wid * ew
        pltpu.sync_copy(z_h, acc.at[pl.ds(sid * rows_per_tile, rows_per_tile)])
        plsc.subcore_barrier()

        def issue(j, b):
            off = base + j * CH
            pltpu.async_copy(m_h.at[pl.ds(off, CH)], mr[b], semm[b])
            pltpu.async_copy(dst_h.at[pl.ds(off, CH)], ix[b], semix[b])

        issue(0, 0)
        issue(1, 1)

        @pl.loop(0, npairs)
        def _(jj):
            for b in range(2):
                j = 2 * jj + b
                pltpu.make_async_copy(m_h.at[pl.ds(base, CH)], mr[b],
                                      semm[b]).wait()
                pltpu.make_async_copy(dst_h.at[pl.ds(base, CH)], ix[b],
                                      semix[b]).wait()
                pltpu.sync_copy(mr[b], acc.at[ix[b]], add=True)

                @pl.when(jj + 1 < npairs)
                def _():
                    issue(j + 2, b)

        off = base + nfull * CH
        pltpu.sync_copy(m_h.at[pl.ds(off, tail)], mr_t)
        pltpu.sync_copy(dst_h.at[pl.ds(off, tail)], ix_t)
        pltpu.sync_copy(mr_t, acc.at[ix_t], add=True)

        plsc.subcore_barrier()
        pltpu.sync_copy(
            acc.at[pl.ds(sid * rows_per_tile, rows_per_tile)],
            out_h.at[pl.ds(cid * n + sid * rows_per_tile, rows_per_tile)])

    return run(m, dst, zeros)


def _place(shape, blocks):
    out = jnp.zeros(shape, F32)
    for (r, c, blk) in blocks:
        out = lax.dynamic_update_slice(out, blk.astype(F32), (r, c))
    return out


def kernel(coords, atoms, masses, edge_index, batch_ptrs, moments, t, params):
    p = params
    n = coords.shape[0]
    e = edge_index.shape[1]
    bsz = moments.shape[0]
    nper = n // bsz
    nl = len(p['blocks'])

    src = edge_index[0].astype(jnp.int32)
    dst = edge_index[1].astype(jnp.int32)
    atoms2 = atoms.reshape(n, 1).astype(jnp.int32)
    n_pad = ((n // 16 + 7) // 8 * 8) * 16
    zeros = jnp.zeros((n_pad // 16, 128), F32)

    w1, b1, w2, b2 = p['proj_node']
    h = _node_init(coords, atoms2, masses, p['emb_atom'],
                   w1[0:3], w1[3:35], w1[35:36],
                   b1.reshape(1, -1), w2, b2.reshape(1, -1))

    cw1, cb1, cw2, cb2 = p['proj_cond']
    wcs = [bp['Wc'] for bp in p['blocks']] + [p['head_norm'][0]]
    bcs = [bp['bc'] for bp in p['blocks']] + [p['head_norm'][1]]
    gws = jnp.stack([w[:, :64] for w in wcs])
    bws = jnp.stack([w[:, 64:] for w in wcs])
    bgs = jnp.stack([b[:64].reshape(1, 64) for b in bcs])
    bbs = jnp.stack([b[64:].reshape(1, 64) for b in bcs])
    def sinus(x, lo, hi):
        waves = jnp.asarray(np.geomspace(lo, hi, 32), F32)
        ang = x[..., None] * (2.0 * np.pi / waves)
        return jnp.concatenate([jnp.sin(ang), jnp.cos(ang)], axis=-1).reshape(
            x.shape[0], -1)

    x256 = jnp.concatenate([sinus(t, 0.001, 1.0), sinus(moments, 1e-4, 1e4)],
                           axis=1)
    ga_l, ba_l = _cond(x256, moments, cw1[0:256], cw1[256:259],
                       cb1.reshape(1, -1), cw2, cb2.reshape(1, -1),
                       gws, bws, bgs, bbs)

    r64 = np.zeros((8, 64), np.float32)
    for hh in range(8):
        r64[hh, hh * 8:(hh + 1) * 8] = 1.0
    r64 = jnp.asarray(r64)
    p3 = np.zeros((3, 128), np.float32)
    p3[0:3, 72:75] = np.eye(3, dtype=np.float32)
    p3 = jnp.asarray(p3)
    p3zero = jnp.zeros((3, 128), F32)

    ew1, eb1, ew2, eb2 = p['proj_edge']
    econsts_base = {
        'r64': r64,
        'w1p': ew1, 'b1p': eb1.reshape(1, 3),
        'w2p': ew2, 'b2p': eb2.reshape(1, 16),
    }

    a = None
    for l in range(nl):
        bp = p['blocks'][l]
        wm, wg = bp['Wm'], bp['Wg']
        if l == 0:
            wcat_s = _place((64, 128), [(0, 0, wm[0:64]), (0, 64, wg[0:64])])
            wcat_d = _place((64, 128),
                            [(0, 0, wm[64:128]), (0, 64, wg[64:128])])
            p3l = p3
        else:
            wcat_s = jnp.concatenate([wm[0:64], _dot(wg[0:64], r64)], axis=1)
            wcat_d = jnp.concatenate([wm[64:128], _dot(wg[64:128], r64)],
                                     axis=1)
            p3l = p3zero

        hn, ts, td = _pre(h, ga_l[l], ba_l[l], coords, wcat_s, wcat_d, p3l,
                          nper)
        ss, sd = _sc_gather_pair(ts, td, src, dst)

        consts = dict(econsts_base)
        consts['wm3'] = wm[128:144]
        consts['wg3'] = _dot(wg[128:144], r64)
        consts['bm'] = bp['bm'].reshape(1, 64)
        consts['bg'] = _dot(bp['bg'].reshape(1, 8), r64)
        out_a = 'Wa' in bp
        if out_a:
            consts['wa'] = bp['Wa']
            consts['ba'] = bp['ba'].reshape(1, 16)

        m, a_next = _edge(ss, sd, a, consts, layer0=(l == 0), out_a=out_a)

        agg2 = _sc_scatter(m, dst, zeros).reshape(2, n_pad, 128)[:, :n, :]

        wu1 = bp['Wu1']
        wu1a = _place((128, 256), [(0, 0, wu1[64:128])])
        h = _node_update(h, hn, agg2, wu1[0:64], wu1a,
                         bp['bu1'].reshape(1, -1), bp['Wu2'],
                         bp['bu2'].reshape(1, -1))
        a = a_next

    hw1, hb1, hw2, hb2 = p['head']
    return _head(h, ga_l[nl], ba_l[nl], coords, hw1, hb1.reshape(1, -1),
                 hw2, hb2.reshape(1, -1), nper)
